# baseline (device time: 17898 ns/iter reference)
import jax
import jax.numpy as jnp
from jax import lax
from jax.experimental import pallas as pl
from jax.experimental.pallas import tpu as pltpu

N_DEV = 32
EPS = 1e-5


def kernel(x, t_emb, W_scale, W_shift):
    b, s, c_loc = x.shape
    c_glob = c_loc * N_DEV

    def body(x_hbm, t_hbm, ws_hbm, wsh_hbm, out_ref,
             xv_ref, t_ref, ws_ref, wsh_ref,
             stats_ref, xrecv_ref, sx_ref, yrecv_ref, sxy_ref, zrecv_ref,
             copy_sems, xsend_sem, xrecv_sem,
             ysend_sems, yrecv_sems, zsend_sems, zrecv_sems,
             ready_y, ready_z):
        my = lax.axis_index("i")

        gz = my // 8
        rr = my % 8
        gy = rr // 2
        xb = rr % 2
        gx = jnp.where(gy % 2 == 0, xb, 1 - xb)

        def ring(px, py, pz):
            return pz * 8 + py * 2 + jnp.where(py % 2 == 0, px, 1 - px)

        x_peer = ring(1 - gx, gy, gz)
        y_peers = [ring(gx, (gy + dy) % 4, gz) for dy in (1, 2, 3)]
        z_peers = [ring(gx, gy, (gz + dz) % 4) for dz in (1, 2, 3)]

        barrier = pltpu.get_barrier_semaphore()
        pl.semaphore_signal(barrier, inc=1, device_id=(x_peer,),
                            device_id_type=pl.DeviceIdType.MESH)
        for p in y_peers:
            pl.semaphore_signal(ready_y, inc=1, device_id=(p,),
                                device_id_type=pl.DeviceIdType.MESH)
        for p in z_peers:
            pl.semaphore_signal(ready_z, inc=1, device_id=(p,),
                                device_id_type=pl.DeviceIdType.MESH)

        cp_x = pltpu.make_async_copy(x_hbm, xv_ref, copy_sems.at[0])
        cp_t = pltpu.make_async_copy(t_hbm, t_ref, copy_sems.at[1])
        cp_ws = pltpu.make_async_copy(ws_hbm, ws_ref, copy_sems.at[2])
        cp_wsh = pltpu.make_async_copy(wsh_hbm, wsh_ref, copy_sems.at[3])
        cp_x.start()
        cp_t.start()
        cp_ws.start()
        cp_wsh.start()

        cp_x.wait()
        xv = xv_ref[...]
        stats_ref[0, :, :] = jnp.sum(xv, axis=-1)
        stats_ref[1, :, :] = jnp.sum(xv * xv, axis=-1)

        pl.semaphore_wait(barrier, 1)
        rdma_x = pltpu.make_async_remote_copy(
            src_ref=stats_ref, dst_ref=xrecv_ref,
            send_sem=xsend_sem, recv_sem=xrecv_sem,
            device_id=(x_peer,), device_id_type=pl.DeviceIdType.MESH,
        )
        rdma_x.start()

        cp_t.wait()
        cp_ws.wait()
        cp_wsh.wait()
        scale = jnp.dot(t_ref[...], ws_ref[...],
                        preferred_element_type=jnp.float32)
        shift = jnp.dot(t_ref[...], wsh_ref[...],
                        preferred_element_type=jnp.float32)

        rdma_x.wait_recv()
        sx_ref[...] = stats_ref[...] + xrecv_ref[...]

        pl.semaphore_wait(ready_y, 3)
        rdmas_y = []
        for i, p in enumerate(y_peers):
            r = pltpu.make_async_remote_copy(
                src_ref=sx_ref, dst_ref=yrecv_ref.at[i],
                send_sem=ysend_sems.at[i], recv_sem=yrecv_sems.at[i],
                device_id=(p,), device_id_type=pl.DeviceIdType.MESH,
            )
            r.start()
            rdmas_y.append(r)
        for r in rdmas_y:
            r.wait_recv()
        sxy_ref[...] = sx_ref[...] + jnp.sum(yrecv_ref[...], axis=0)

        pl.semaphore_wait(ready_z, 3)
        rdmas_z = []
        for i, p in enumerate(z_peers):
            r = pltpu.make_async_remote_copy(
                src_ref=sxy_ref, dst_ref=zrecv_ref.at[i],
                send_sem=zsend_sems.at[i], recv_sem=zrecv_sems.at[i],
                device_id=(p,), device_id_type=pl.DeviceIdType.MESH,
            )
            r.start()
            rdmas_z.append(r)
        for r in rdmas_z:
            r.wait_recv()
        tot = sxy_ref[...] + jnp.sum(zrecv_ref[...], axis=0)

        mean = tot[0] * (1.0 / c_glob)
        var = tot[1] * (1.0 / c_glob) - mean * mean
        inv = lax.rsqrt(var + EPS)

        h = (xv - mean[..., None]) * inv[..., None]
        out_ref[...] = h * (1.0 + scale[:, None, :]) + shift[:, None, :]

        rdma_x.wait_send()
        for r in rdmas_y:
            r.wait_send()
        for r in rdmas_z:
            r.wait_send()

    return pl.pallas_call(
        body,
        out_shape=jax.ShapeDtypeStruct((b, s, c_loc), jnp.float32),
        in_specs=[pl.BlockSpec(memory_space=pl.ANY)] * 4,
        out_specs=pl.BlockSpec(memory_space=pltpu.VMEM),
        scratch_shapes=[
            pltpu.VMEM((b, s, c_loc), jnp.float32),
            pltpu.VMEM(t_emb.shape, jnp.float32),
            pltpu.VMEM(W_scale.shape, jnp.float32),
            pltpu.VMEM(W_shift.shape, jnp.float32),
            pltpu.VMEM((2, b, s), jnp.float32),
            pltpu.VMEM((2, b, s), jnp.float32),
            pltpu.VMEM((2, b, s), jnp.float32),
            pltpu.VMEM((3, 2, b, s), jnp.float32),
            pltpu.VMEM((2, b, s), jnp.float32),
            pltpu.VMEM((3, 2, b, s), jnp.float32),
            pltpu.SemaphoreType.DMA((4,)),
            pltpu.SemaphoreType.DMA,
            pltpu.SemaphoreType.DMA,
            pltpu.SemaphoreType.DMA((3,)),
            pltpu.SemaphoreType.DMA((3,)),
            pltpu.SemaphoreType.DMA((3,)),
            pltpu.SemaphoreType.DMA((3,)),
            pltpu.SemaphoreType.REGULAR,
            pltpu.SemaphoreType.REGULAR,
        ],
        compiler_params=pltpu.CompilerParams(collective_id=0),
    )(x, t_emb, W_scale, W_shift)
